# baseline (speedup 1.0000x reference)
import functools

import jax
import jax.numpy as jnp
from jax import lax
from jax.experimental import pallas as pl
from jax.experimental.pallas import tpu as pltpu
from jax.experimental.pallas import tpu_sc as plsc

B, N, M, D = 8, 2048, 2048, 3
B_TC = 7
NW = 32
ROWS = N // NW
RU = 8
BIG = 3.0e38


def _tc_body(t_ref, s_ref, o_ref, acc_ref):
    b = pl.program_id(0)
    t = t_ref[0]
    sT = s_ref[0]
    tn = jnp.sum(t * t, axis=1, keepdims=True)
    sn = 0.25 * jnp.sum(sT * sT, axis=0, keepdims=True)
    prod = jax.lax.dot_general(
        t, sT, (((1,), (0,)), ((), ())),
        preferred_element_type=jnp.float32)
    d = prod + tn + sn
    rowmin = jnp.maximum(jnp.min(d, axis=1), 0.0)
    colmin = jnp.maximum(jnp.min(d, axis=0), 0.0)
    s1 = jnp.sum(jnp.sqrt(rowmin))
    s2 = jnp.sum(jnp.sqrt(colmin))

    @pl.when(b == 0)
    def _init():
        acc_ref[0] = 0.0
        acc_ref[1] = 0.0

    acc_ref[0] += s1
    acc_ref[1] += s2

    @pl.when(b == B_TC - 1)
    def _fin():
        o_ref[0, 0] = acc_ref[0]
        o_ref[0, 1] = acc_ref[1]


_sc_mesh = plsc.VectorSubcoreMesh(core_axis_name="c", subcore_axis_name="s")


@functools.partial(
    pl.kernel,
    out_type=[
        jax.ShapeDtypeStruct((NW * M,), jnp.float32),
        jax.ShapeDtypeStruct((N * 16,), jnp.float32),
    ],
    mesh=_sc_mesh,
    scratch_types=[
        pltpu.VMEM((D * N + 16,), jnp.float32),
        pltpu.VMEM((D * M,), jnp.float32),
        pltpu.VMEM((M,), jnp.float32),
        pltpu.VMEM((ROWS * 16,), jnp.float32),
    ],
)
def _sc_kernel(t_hbm, s_hbm, ocol_hbm, orow_hbm, t_vm, s_vm, col_vm, row_vm):
    w = lax.axis_index("s") * 2 + lax.axis_index("c")
    pltpu.sync_copy(t_hbm, t_vm.at[pl.ds(0, D * N)])
    pltpu.sync_copy(s_hbm, s_vm)

    def init_col(j, carry):
        col_vm[pl.ds(j * 16, 16)] = jnp.full((16,), BIG, jnp.float32)
        return carry

    lax.fori_loop(0, M // 16, init_col, 0)

    row0 = w * ROWS

    def row_block(rb, carry):
        base = row0 + rb * RU
        tx = [None] * RU
        ty = [None] * RU
        tz = [None] * RU
        vx = t_vm[pl.ds(base, 16)]
        vy = t_vm[pl.ds(N + base, 16)]
        vz = t_vm[pl.ds(2 * N + base, 16)]
        for k in range(RU):
            tx[k] = jnp.full((16,), vx[k], jnp.float32)
            ty[k] = jnp.full((16,), vy[k], jnp.float32)
            tz[k] = jnp.full((16,), vz[k], jnp.float32)

        def src_chunk(j, racc):
            sx = s_vm[pl.ds(j * 16, 16)]
            sy = s_vm[pl.ds(M + j * 16, 16)]
            sz = s_vm[pl.ds(2 * M + j * 16, 16)]
            cacc = col_vm[pl.ds(j * 16, 16)]
            out = []
            for k in range(RU):
                dx = sx - tx[k]
                dy = sy - ty[k]
                dz = sz - tz[k]
                dd = dx * dx + dy * dy + dz * dz
                out.append(jnp.minimum(racc[k], dd))
                cacc = jnp.minimum(cacc, dd)
            col_vm[pl.ds(j * 16, 16)] = cacc
            return tuple(out)

        racc = lax.fori_loop(
            0, M // 16, src_chunk,
            tuple(jnp.full((16,), BIG, jnp.float32) for _ in range(RU)))
        for k in range(RU):
            row_vm[pl.ds((rb * RU + k) * 16, 16)] = racc[k]
        return carry

    lax.fori_loop(0, ROWS // RU, row_block, 0)

    pltpu.sync_copy(col_vm, ocol_hbm.at[pl.ds(w * M, M)])
    pltpu.sync_copy(row_vm, orow_hbm.at[pl.ds(row0 * 16, ROWS * 16)])


def kernel(template, source):
    t_flat = jnp.swapaxes(template[B_TC], 0, 1).reshape(D * N)
    s_flat = jnp.swapaxes(source[B_TC], 0, 1).reshape(D * M)
    col_part, row_part = _sc_kernel(t_flat, s_flat)

    sT = jnp.swapaxes(source[:B_TC], 1, 2) * -2.0
    tc_out = pl.pallas_call(
        _tc_body,
        grid=(B_TC,),
        in_specs=[
            pl.BlockSpec((1, N, D), lambda b: (b, 0, 0)),
            pl.BlockSpec((1, D, M), lambda b: (b, 0, 0)),
        ],
        out_specs=pl.BlockSpec(memory_space=pltpu.SMEM),
        out_shape=jax.ShapeDtypeStruct((1, 2), jnp.float32),
        scratch_shapes=[pltpu.SMEM((2,), jnp.float32)],
    )(template[:B_TC], sT)

    col_sc = jnp.maximum(jnp.min(col_part.reshape(NW, M), axis=0), 0.0)
    row_sc = jnp.maximum(jnp.min(row_part.reshape(N, 16), axis=1), 0.0)
    s1 = tc_out[0, 0] + jnp.sum(jnp.sqrt(row_sc))
    s2 = tc_out[0, 1] + jnp.sum(jnp.sqrt(col_sc))
    return (s1 / (B * N) + s2 / (B * M)) * 0.5


# speedup vs baseline: 1.6741x; 1.6741x over previous
import jax
import jax.numpy as jnp
from jax.experimental import pallas as pl
from jax.experimental.pallas import tpu as pltpu

B, N, M, D = 8, 2048, 2048, 3


def _chamfer_body(t_ref, s_ref, o_ref, acc_ref):
    b = pl.program_id(0)
    t = t_ref[0]
    sT = s_ref[0]
    tn = jnp.sum(t * t, axis=1, keepdims=True)
    sn = 0.25 * jnp.sum(sT * sT, axis=0, keepdims=True)
    prod = jax.lax.dot_general(
        t, sT, (((1,), (0,)), ((), ())),
        preferred_element_type=jnp.float32)
    d = prod + tn + sn
    rowmin = jnp.maximum(jnp.min(d, axis=1), 0.0)
    colmin = jnp.maximum(jnp.min(d, axis=0), 0.0)
    s1 = jnp.sum(jnp.sqrt(rowmin))
    s2 = jnp.sum(jnp.sqrt(colmin))

    @pl.when(b == 0)
    def _init():
        acc_ref[0] = 0.0
        acc_ref[1] = 0.0

    acc_ref[0] += s1
    acc_ref[1] += s2

    @pl.when(b == B - 1)
    def _fin():
        c1 = acc_ref[0] / (B * N)
        c2 = acc_ref[1] / (B * M)
        o_ref[0, 0] = (c1 + c2) * 0.5


def kernel(template, source):
    sT = jnp.swapaxes(source, 1, 2) * -2.0
    out = pl.pallas_call(
        _chamfer_body,
        grid=(B,),
        in_specs=[
            pl.BlockSpec((1, N, D), lambda b: (b, 0, 0)),
            pl.BlockSpec((1, D, M), lambda b: (b, 0, 0)),
        ],
        out_specs=pl.BlockSpec(memory_space=pltpu.SMEM),
        out_shape=jax.ShapeDtypeStruct((1, 1), jnp.float32),
        scratch_shapes=[pltpu.SMEM((2,), jnp.float32)],
    )(template, sT)
    return out[0, 0]
